# Initial kernel scaffold; baseline (speedup 1.0000x reference)
import jax
import jax.numpy as jnp
from jax.experimental import pallas as pl


def kernel(batch, ent_embs, rel_embs):
    raise NotImplementedError("write your pallas kernel here")


# speedup vs baseline: 1.8732x; 1.8732x over previous
import functools

import jax
import jax.numpy as jnp
from jax import lax
from jax.experimental import pallas as pl
from jax.experimental.pallas import tpu as pltpu
from jax.experimental.pallas import tpu_sc as plsc

NC = 2
NS = 16
L = 16
NW = NC * NS
B = 16384
D = 128
BPW = B // NW
CH = 128
NCHUNK = BPW // CH
GROUPS = CH // L


def _nsqrt(x):
    i = plsc.bitcast(x, jnp.int32)
    i = jnp.int32(0x1FBD1DF5) + lax.shift_right_logical(i, jnp.int32(1))
    y = plsc.bitcast(i, jnp.float32)
    for _ in range(3):
        y = 0.5 * (y + x / y)
    return y


@functools.partial(
    pl.kernel,
    out_type=jax.ShapeDtypeStruct((B,), jnp.float32),
    mesh=plsc.VectorSubcoreMesh(
        core_axis_name="c", subcore_axis_name="s", num_cores=NC, num_subcores=NS
    ),
    compiler_params=pltpu.CompilerParams(needs_layout_passes=False),
    scratch_types=[
        pltpu.VMEM((3, NCHUNK, CH), jnp.int32),
        pltpu.VMEM((CH, D), jnp.float32),
        pltpu.VMEM((CH, D), jnp.float32),
        pltpu.VMEM((CH, D), jnp.float32),
        pltpu.VMEM((CH * L,), jnp.float32),
        pltpu.VMEM((BPW,), jnp.float32),
        pltpu.SemaphoreType.DMA,
    ],
)
def _sc_score(
    idx_hbm, ent_hbm, rel_hbm, out_hbm, idx_v, bh, br, bt, pvec, outv, sem
):
    c = lax.axis_index("c")
    s = lax.axis_index("s")
    wid = s * NC + c
    pltpu.sync_copy(idx_hbm.at[wid], idx_v)
    for ch in range(NCHUNK):
        cp_h = pltpu.async_copy(ent_hbm.at[idx_v.at[0, ch]], bh, sem)
        cp_r = pltpu.async_copy(rel_hbm.at[idx_v.at[1, ch]], br, sem)
        cp_t = pltpu.async_copy(ent_hbm.at[idx_v.at[2, ch]], bt, sem)
        cp_h.wait()
        cp_r.wait()
        cp_t.wait()

        def item(i, carry):
            acc = jnp.zeros((L,), jnp.float32)
            for j in range(D // L):
                h = bh[i, pl.ds(j * L, L)]
                r = br[i, pl.ds(j * L, L)]
                t = bt[i, pl.ds(j * L, L)]
                d = (h + r) - t
                acc = acc + d * d
            pvec[pl.ds(i * L, L)] = acc
            return carry

        lax.fori_loop(0, CH, item, 0)

        def group(g, carry, ch=ch):
            base = (g * L + lax.iota(jnp.int32, L)) * L
            tot = jnp.zeros((L,), jnp.float32)
            for j in range(L):
                tot = tot + plsc.load_gather(pvec, [base + j])
            score = -_nsqrt(tot)
            oidx = ch * CH + g * L + lax.iota(jnp.int32, L)
            plsc.store_scatter(outv, [oidx], score)
            return carry

        lax.fori_loop(0, GROUPS, group, 0)
    pltpu.sync_copy(outv, out_hbm.at[pl.ds(wid * BPW, BPW)])


def kernel(batch, ent_embs, rel_embs):
    idx = (
        batch.astype(jnp.int32)
        .T.reshape(3, NW, NCHUNK, CH)
        .transpose(1, 0, 2, 3)
    )
    scores = _sc_score(idx, ent_embs, rel_embs)
    return scores.reshape(-1, 1)


# speedup vs baseline: 2.2566x; 1.2047x over previous
import functools

import jax
import jax.numpy as jnp
from jax import lax
from jax.experimental import pallas as pl
from jax.experimental.pallas import tpu as pltpu
from jax.experimental.pallas import tpu_sc as plsc

NC = 2
NS = 16
L = 16
NW = NC * NS
B = 16384
D = 128
BPW = B // NW
CH = 128
NCHUNK = BPW // CH
GROUPS = CH // L


def _nsqrt(x):
    i = plsc.bitcast(x, jnp.int32)
    i = jnp.int32(0x1FBD1DF5) + lax.shift_right_logical(i, jnp.int32(1))
    y = plsc.bitcast(i, jnp.float32)
    for _ in range(3):
        y = 0.5 * (y + x / y)
    return y


@functools.partial(
    pl.kernel,
    out_type=jax.ShapeDtypeStruct((B,), jnp.float32),
    mesh=plsc.VectorSubcoreMesh(
        core_axis_name="c", subcore_axis_name="s", num_cores=NC, num_subcores=NS
    ),
    compiler_params=pltpu.CompilerParams(needs_layout_passes=False),
    scratch_types=[
        pltpu.VMEM((3, NCHUNK, CH), jnp.int32),
        [pltpu.VMEM((CH, D), jnp.float32) for _ in range(2)],
        [pltpu.VMEM((CH, D), jnp.float32) for _ in range(2)],
        [pltpu.VMEM((CH, D), jnp.float32) for _ in range(2)],
        pltpu.VMEM((CH * L,), jnp.float32),
        pltpu.VMEM((BPW,), jnp.float32),
        [pltpu.SemaphoreType.DMA for _ in range(2)],
    ],
)
def _sc_score(
    idx_hbm, ent_hbm, rel_hbm, out_hbm, idx_v, bhs, brs, bts, pvec, outv, sems
):
    c = lax.axis_index("c")
    s = lax.axis_index("s")
    wid = s * NC + c
    pltpu.sync_copy(idx_hbm.at[wid], idx_v)

    def fire(ch):
        b = ch % 2
        return (
            pltpu.async_copy(ent_hbm.at[idx_v.at[0, ch]], bhs[b], sems[b]),
            pltpu.async_copy(rel_hbm.at[idx_v.at[1, ch]], brs[b], sems[b]),
            pltpu.async_copy(ent_hbm.at[idx_v.at[2, ch]], bts[b], sems[b]),
        )

    pending = fire(0)
    for ch in range(NCHUNK):
        for cp in pending:
            cp.wait()
        if ch + 1 < NCHUNK:
            pending = fire(ch + 1)
        bh, br, bt = bhs[ch % 2], brs[ch % 2], bts[ch % 2]

        def item(i, carry, bh=bh, br=br, bt=bt):
            acc = jnp.zeros((L,), jnp.float32)
            for j in range(D // L):
                h = bh[i, pl.ds(j * L, L)]
                r = br[i, pl.ds(j * L, L)]
                t = bt[i, pl.ds(j * L, L)]
                d = (h + r) - t
                acc = acc + d * d
            pvec[pl.ds(i * L, L)] = acc
            return carry

        lax.fori_loop(0, CH, item, 0)

        def group(g, carry, ch=ch):
            base = (g * L + lax.iota(jnp.int32, L)) * L
            tot = jnp.zeros((L,), jnp.float32)
            for j in range(L):
                tot = tot + plsc.load_gather(pvec, [base + j])
            score = -_nsqrt(tot)
            oidx = ch * CH + g * L + lax.iota(jnp.int32, L)
            plsc.store_scatter(outv, [oidx], score)
            return carry

        lax.fori_loop(0, GROUPS, group, 0)
    pltpu.sync_copy(outv, out_hbm.at[pl.ds(wid * BPW, BPW)])


def kernel(batch, ent_embs, rel_embs):
    idx = (
        batch.astype(jnp.int32)
        .T.reshape(3, NW, NCHUNK, CH)
        .transpose(1, 0, 2, 3)
    )
    scores = _sc_score(idx, ent_embs, rel_embs)
    return scores.reshape(-1, 1)
